# baseline (speedup 1.0000x reference)
import functools

import jax
import jax.numpy as jnp
import numpy as np
from jax import lax
from jax.experimental import pallas as pl
from jax.experimental.pallas import tpu as pltpu
from jax.experimental.pallas import tpu_sc as plsc

_FIELD = 38462
_F = 26
_E = 16
_B = 16384
_NC = 2
_NS = 16
_NW = _NC * _NS
_RPW = _B // _NW
_CHUNK = 64
_NCHUNK = _RPW // _CHUNK


def _sc_body(x_hbm, w_hbm, out_hbm, w_v, x_v0, x_v1, o_v0, o_v1, xsem, osem):
    wid = lax.axis_index("s") * _NC + lax.axis_index("c")
    pltpu.sync_copy(w_hbm, w_v)
    base = wid * _RPW
    xbufs = [x_v0, x_v1]
    obufs = [o_v0, o_v1]

    def x_copy(c):
        lo = base + c * _CHUNK
        return pltpu.async_copy(
            x_hbm.at[pl.ds(lo, _CHUNK), :], xbufs[c % 2], xsem.at[c % 2]
        )

    def o_copy(c):
        lo = base + c * _CHUNK
        return pltpu.async_copy(
            obufs[c % 2], out_hbm.at[pl.ds(lo, _CHUNK), :], osem.at[c % 2]
        )

    xcp = {0: x_copy(0)}
    ocp = {}
    for c in range(_NCHUNK):
        if c + 1 < _NCHUNK:
            xcp[c + 1] = x_copy(c + 1)
        xcp[c].wait()
        if c >= 2:
            ocp[c - 2].wait()
        x_v = xbufs[c % 2]
        o_v = obufs[c % 2]

        @plsc.parallel_loop(0, _CHUNK, 1, unroll=3)
        def _row(i):
            bi = jnp.broadcast_to(i, (_E,))
            for f in range(_F):
                bf = jnp.full((_E,), f, jnp.int32)
                xi = plsc.load_gather(x_v, [bi, bf])
                o_v[i, pl.ds(f * _E, _E)] = xi * w_v[f]

        ocp[c] = o_copy(c)
    ocp[_NCHUNK - 2].wait()
    ocp[_NCHUNK - 1].wait()


@jax.jit
def kernel(x, weight):
    offsets = jnp.asarray(np.arange(_F, dtype=np.int32) * _FIELD)
    w26 = jnp.take(weight, offsets, axis=0)
    mesh = plsc.VectorSubcoreMesh(core_axis_name="c", subcore_axis_name="s")
    run = functools.partial(
        pl.kernel,
        mesh=mesh,
        out_type=jax.ShapeDtypeStruct((_B, _F * _E), jnp.float32),
        scratch_types=[
            pltpu.VMEM((_F, _E), jnp.float32),
            pltpu.VMEM((_CHUNK, _F), jnp.float32),
            pltpu.VMEM((_CHUNK, _F), jnp.float32),
            pltpu.VMEM((_CHUNK, _F * _E), jnp.float32),
            pltpu.VMEM((_CHUNK, _F * _E), jnp.float32),
            pltpu.SemaphoreType.DMA((2,)),
            pltpu.SemaphoreType.DMA((2,)),
        ],
        compiler_params=pltpu.CompilerParams(needs_layout_passes=False),
    )(_sc_body)
    return run(x.astype(jnp.float32), w26)


# speedup vs baseline: 1.1763x; 1.1763x over previous
import functools

import jax
import jax.numpy as jnp
import numpy as np
from jax import lax
from jax.experimental import pallas as pl
from jax.experimental.pallas import tpu as pltpu
from jax.experimental.pallas import tpu_sc as plsc

_FIELD = 38462
_F = 26
_E = 16
_B = 16384
_NC = 2
_NS = 16
_NW = _NC * _NS
_RPW = _B // _NW
_CHUNK = 64
_NCHUNK = _RPW // _CHUNK


def _sc_body(x_hbm, w_hbm, out_hbm, w_v, x_v0, x_v1, o_v0, o_v1, xsem, osem):
    wid = lax.axis_index("s") * _NC + lax.axis_index("c")
    pltpu.sync_copy(w_hbm, w_v)
    base = wid * _RPW
    xbufs = [x_v0, x_v1]
    obufs = [o_v0, o_v1]

    def x_copy(c):
        lo = base + c * _CHUNK
        return pltpu.async_copy(
            x_hbm.at[pl.ds(lo, _CHUNK), :], xbufs[c % 2], xsem.at[c % 2]
        )

    def o_copy(c):
        lo = base + c * _CHUNK
        return pltpu.async_copy(
            obufs[c % 2], out_hbm.at[pl.ds(lo, _CHUNK), :], osem.at[c % 2]
        )

    xcp = {0: x_copy(0)}
    ocp = {}
    for c in range(_NCHUNK):
        if c + 1 < _NCHUNK:
            xcp[c + 1] = x_copy(c + 1)
        xcp[c].wait()
        if c >= 2:
            ocp[c - 2].wait()
        x_v = xbufs[c % 2]
        o_v = obufs[c % 2]

        @plsc.parallel_loop(0, _CHUNK, 1, unroll=2)
        def _row(i):
            bi = jnp.broadcast_to(i, (_E,))
            for f in range(_F):
                bf = jnp.full((_E,), f, jnp.int32)
                xi = plsc.load_gather(x_v, [bi, bf])
                o_v[i, pl.ds(f * _E, _E)] = xi * w_v[f]

        ocp[c] = o_copy(c)
    ocp[_NCHUNK - 2].wait()
    ocp[_NCHUNK - 1].wait()


@jax.jit
def kernel(x, weight):
    offsets = jnp.asarray(np.arange(_F, dtype=np.int32) * _FIELD)
    w26 = jnp.take(weight, offsets, axis=0)
    mesh = plsc.VectorSubcoreMesh(core_axis_name="c", subcore_axis_name="s")
    run = functools.partial(
        pl.kernel,
        mesh=mesh,
        out_type=jax.ShapeDtypeStruct((_B, _F * _E), jnp.float32),
        scratch_types=[
            pltpu.VMEM((_F, _E), jnp.float32),
            pltpu.VMEM((_CHUNK, _F), jnp.float32),
            pltpu.VMEM((_CHUNK, _F), jnp.float32),
            pltpu.VMEM((_CHUNK, _F * _E), jnp.float32),
            pltpu.VMEM((_CHUNK, _F * _E), jnp.float32),
            pltpu.SemaphoreType.DMA((2,)),
            pltpu.SemaphoreType.DMA((2,)),
        ],
        compiler_params=pltpu.CompilerParams(needs_layout_passes=False),
    )(_sc_body)
    return run(x.astype(jnp.float32), w26)
